# baseline (speedup 1.0000x reference)
import functools

import jax
import jax.numpy as jnp
from jax import lax
from jax.experimental import pallas as pl
from jax.experimental.pallas import tpu as pltpu
from jax.experimental.pallas import tpu_sc as plsc

N = 10000
E = 320000
D_IN = 128
H = 20
HP = 32
DEC_H = 64
N_PAD = 10240
NC = 2
NS = 16
NW = NC * NS
CHUNK = 128
NCH = 80
SUP = 4
TOT_CHUNKS = NW * NCH
E_PAD = TOT_CHUNKS * CHUNK
SLACK = 128
DEG_SPLIT = (80, 80)
SCAT_SPLIT = (80, 80)
GATH_SPLIT = (80, 80)
NE4 = E_PAD // 4
ROWS_PER_TILE = N_PAD // NS

def _make_noise_u4():
    import numpy as np

    def rotl(x, r):
        return ((x << np.uint32(r)) | (x >> np.uint32(32 - r))).astype(np.uint32)

    rotations = ((13, 15, 26, 6), (17, 29, 16, 24))
    ks0, ks1 = np.uint32(0), np.uint32(42)
    ks2 = np.uint32(ks0 ^ ks1 ^ np.uint32(0x1BD11BDA))
    ks = (ks0, ks1, ks2)
    i = np.arange(E, dtype=np.uint64)
    x0 = (i >> np.uint64(32)).astype(np.uint32)
    x1 = (i & np.uint64(0xFFFFFFFF)).astype(np.uint32)
    x0 = (x0 + ks0).astype(np.uint32)
    x1 = (x1 + ks1).astype(np.uint32)
    for r in range(5):
        for rot in rotations[r % 2]:
            x0 = (x0 + x1).astype(np.uint32)
            x1 = rotl(x1, rot)
            x1 = (x1 ^ x0).astype(np.uint32)
        x0 = (x0 + ks[(r + 1) % 3]).astype(np.uint32)
        x1 = (x1 + ks[(r + 2) % 3] + np.uint32(r + 1)).astype(np.uint32)
    bits = (x0 ^ x1).astype(np.uint32)
    floats = (bits >> np.uint32(9)) | np.uint32(0x3F800000)
    u = floats.view(np.float32) - np.float32(1.0)
    u_pad = np.full((E_PAD,), 0.5, np.float32)
    u_pad[:E] = u
    return np.ascontiguousarray(u_pad.reshape(NE4, 4).T)


_U4 = _make_noise_u4()


def _f32(*shape):
    return jax.ShapeDtypeStruct(shape, jnp.float32)


@functools.lru_cache(maxsize=None)
def _mesh():
    return plsc.VectorSubcoreMesh(core_axis_name="c", subcore_axis_name="s",
                                  num_cores=NC, num_subcores=NS)


def _pipe2(nch, start_fn, finish_fn):
    start_fn(0, 0)

    def body(i, _):
        j0 = 2 * i
        j1 = j0 + 1
        start_fn(j1, 1)
        finish_fn(j0, 0)

        @pl.when(j1 + 1 < nch)
        def _():
            start_fn(j1 + 1, 0)

        finish_fn(j1, 1)
        return 0

    lax.fori_loop(0, nch // 2, body, 0)


def _sc_degree(col3, ones_t, zeros_t):
    return _sc_degree_kernel()(col3, ones_t, zeros_t)


@functools.lru_cache(maxsize=None)
def _sc_degree_kernel():
    return functools.partial(
        pl.kernel,
        out_type=_f32(NC, N_PAD, HP),
        mesh=_mesh(),
        compiler_params=pltpu.CompilerParams(use_tc_tiling_on_sc=False),
        scratch_types=[
            pltpu.VMEM((max(DEG_SPLIT), CHUNK), jnp.int32),
            pltpu.VMEM((CHUNK, HP), jnp.float32),
            pltpu.VMEM_SHARED((N_PAD, HP), jnp.float32),
        ],
    )(_sc_degree_body)


def _split_range(cid, sid, split):
    n0, n1 = split
    if n0 == n1:
        return (cid * NS + sid) * n0, n0
    start = jnp.where(cid == 0, sid * n0, NS * n0 + sid * n1)
    nch = jnp.where(cid == 0, n0, n1)
    return start, nch


def _sc_degree_body(col_hbm, ones_hbm, zeros_hbm, out_hbm, cidx, ones_v, hist_sh):
    cid = lax.axis_index("c")
    sid = lax.axis_index("s")
    start, my_nch = _split_range(cid, sid, DEG_SPLIT)
    pltpu.sync_copy(zeros_hbm.at[pl.ds(sid * ROWS_PER_TILE, ROWS_PER_TILE)],
                    hist_sh.at[pl.ds(sid * ROWS_PER_TILE, ROWS_PER_TILE)])
    pltpu.sync_copy(ones_hbm, ones_v)
    pltpu.sync_copy(col_hbm.at[pl.ds(start, max(DEG_SPLIT))], cidx)
    plsc.subcore_barrier()

    def body(j, _):
        pltpu.sync_copy(ones_v, hist_sh.at[cidx.at[j]], add=True)
        return 0

    lax.fori_loop(0, my_nch, body, 0)
    plsc.subcore_barrier()
    pltpu.sync_copy(hist_sh.at[pl.ds(sid * ROWS_PER_TILE, ROWS_PER_TILE)],
                    out_hbm.at[cid, pl.ds(sid * ROWS_PER_TILE, ROWS_PER_TILE)])


def _sc_scatter(row3, col3, y, zeros_t):
    return _sc_scatter_kernel()(row3, col3, y, zeros_t)


@functools.lru_cache(maxsize=None)
def _sc_scatter_kernel():
    return functools.partial(
        pl.kernel,
        out_type=_f32(NC, N_PAD, HP),
        mesh=_mesh(),
        compiler_params=pltpu.CompilerParams(use_tc_tiling_on_sc=False),
        scratch_types=[
            pltpu.VMEM((max(SCAT_SPLIT), CHUNK), jnp.int32),
            pltpu.VMEM((max(SCAT_SPLIT), CHUNK), jnp.int32),
            pltpu.VMEM((SUP * CHUNK, HP), jnp.float32),
            pltpu.VMEM((SUP * CHUNK, HP), jnp.float32),
            pltpu.VMEM_SHARED((N_PAD, HP), jnp.float32),
            pltpu.SemaphoreType.DMA,
            pltpu.SemaphoreType.DMA,
        ],
    )(_sc_scatter_body)


def _sc_scatter_body(row_hbm, col_hbm, y_hbm, zeros_hbm, out_hbm,
                     ridx, cidx, m0, m1, acc_sh, sg0, sg1):
    cid = lax.axis_index("c")
    sid = lax.axis_index("s")
    start, my_nch = _split_range(cid, sid, SCAT_SPLIT)
    pltpu.sync_copy(zeros_hbm.at[pl.ds(sid * ROWS_PER_TILE, ROWS_PER_TILE)],
                    acc_sh.at[pl.ds(sid * ROWS_PER_TILE, ROWS_PER_TILE)])
    pltpu.sync_copy(row_hbm.at[pl.ds(start, max(SCAT_SPLIT))], ridx)
    pltpu.sync_copy(col_hbm.at[pl.ds(start, max(SCAT_SPLIT))], cidx)
    plsc.subcore_barrier()

    bufs = (m0, m1)
    sems = (sg0, sg1)

    def start(su, slot):
        for c in range(SUP):
            pltpu.async_copy(y_hbm.at[ridx.at[su * SUP + c]],
                             bufs[slot].at[pl.ds(c * CHUNK, CHUNK)],
                             sems[slot])

    def finish(su, slot):
        pltpu.make_async_copy(y_hbm, bufs[slot], sems[slot]).wait()
        for c in range(SUP):
            pltpu.sync_copy(bufs[slot].at[pl.ds(c * CHUNK, CHUNK)],
                            acc_sh.at[cidx.at[su * SUP + c]], add=True)

    _pipe2(my_nch // SUP, start, finish)
    plsc.subcore_barrier()
    pltpu.sync_copy(acc_sh.at[pl.ds(sid * ROWS_PER_TILE, ROWS_PER_TILE)],
                    out_hbm.at[cid, pl.ds(sid * ROWS_PER_TILE, ROWS_PER_TILE)])


def _sc_gather(row3, col3, x1):
    return _sc_gather_kernel()(row3, col3, x1)


@functools.lru_cache(maxsize=None)
def _sc_gather_kernel():
    return functools.partial(
        pl.kernel,
        out_type=(_f32(E_PAD, HP), _f32(E_PAD, HP)),
        mesh=_mesh(),
        compiler_params=pltpu.CompilerParams(use_tc_tiling_on_sc=False),
        scratch_types=[
            pltpu.VMEM((max(GATH_SPLIT), CHUNK), jnp.int32),
            pltpu.VMEM((max(GATH_SPLIT), CHUNK), jnp.int32),
            pltpu.VMEM((SUP * CHUNK, HP), jnp.float32),
            pltpu.VMEM((SUP * CHUNK, HP), jnp.float32),
            pltpu.VMEM((SUP * CHUNK, HP), jnp.float32),
            pltpu.VMEM((SUP * CHUNK, HP), jnp.float32),
            pltpu.SemaphoreType.DMA,
            pltpu.SemaphoreType.DMA,
            pltpu.SemaphoreType.DMA,
            pltpu.SemaphoreType.DMA,
            pltpu.SemaphoreType.DMA,
            pltpu.SemaphoreType.DMA,
            pltpu.SemaphoreType.DMA,
            pltpu.SemaphoreType.DMA,
        ],
    )(_sc_gather_body)


def _sc_gather_body(row_hbm, col_hbm, x1_hbm, za_hbm, zb_hbm,
                    ridx, cidx, a0, a1, b0, b1,
                    sa0, sa1, sb0, sb1, swa0, swa1, swb0, swb1):
    cid = lax.axis_index("c")
    sid = lax.axis_index("s")
    start_ch, my_nch = _split_range(cid, sid, GATH_SPLIT)
    pltpu.sync_copy(row_hbm.at[pl.ds(start_ch, max(GATH_SPLIT))], ridx)
    pltpu.sync_copy(col_hbm.at[pl.ds(start_ch, max(GATH_SPLIT))], cidx)

    abufs = (a0, a1)
    bbufs = (b0, b1)
    asems = (sa0, sa1)
    bsems = (sb0, sb1)
    wasems = (swa0, swa1)
    wbsems = (swb0, swb1)
    sup_rows = SUP * CHUNK

    def start(su, slot):
        @pl.when(su >= 2)
        def _():
            pltpu.make_async_copy(abufs[slot], za_hbm.at[pl.ds(0, sup_rows)],
                                  wasems[slot]).wait()
            pltpu.make_async_copy(bbufs[slot], zb_hbm.at[pl.ds(0, sup_rows)],
                                  wbsems[slot]).wait()
        for c in range(SUP):
            pltpu.async_copy(x1_hbm.at[ridx.at[su * SUP + c]],
                             abufs[slot].at[pl.ds(c * CHUNK, CHUNK)],
                             asems[slot])
            pltpu.async_copy(x1_hbm.at[cidx.at[su * SUP + c]],
                             bbufs[slot].at[pl.ds(c * CHUNK, CHUNK)],
                             bsems[slot])

    def finish(su, slot):
        pltpu.make_async_copy(x1_hbm, abufs[slot], asems[slot]).wait()
        pltpu.make_async_copy(x1_hbm, bbufs[slot], bsems[slot]).wait()
        base = start_ch * CHUNK + su * sup_rows
        pltpu.async_copy(abufs[slot], za_hbm.at[pl.ds(base, sup_rows)],
                         wasems[slot])
        pltpu.async_copy(bbufs[slot], zb_hbm.at[pl.ds(base, sup_rows)],
                         wbsems[slot])

    _pipe2(my_nch // SUP, start, finish)
    for slot in (0, 1):
        pltpu.make_async_copy(abufs[slot], za_hbm.at[pl.ds(0, sup_rows)],
                              wasems[slot]).wait()
        pltpu.make_async_copy(bbufs[slot], zb_hbm.at[pl.ds(0, sup_rows)],
                              wbsems[slot]).wait()


def _tc_y_body(x_ref, w_ref, deg_ref, y_ref):
    deg = deg_ref[0, :, 0:1] + deg_ref[1, :, 0:1] + 1.0
    s = lax.rsqrt(deg)
    xw = jnp.dot(x_ref[...], w_ref[...], preferred_element_type=jnp.float32)
    y_ref[...] = xw * s


def _tc_y(x_pad, wgc_pad, degp):
    R = 1280
    return pl.pallas_call(
        _tc_y_body,
        grid=(N_PAD // R,),
        in_specs=[
            pl.BlockSpec((R, D_IN), lambda i: (i, 0)),
            pl.BlockSpec((D_IN, HP), lambda i: (0, 0)),
            pl.BlockSpec((NC, R, HP), lambda i: (0, i, 0)),
        ],
        out_specs=pl.BlockSpec((R, HP), lambda i: (i, 0)),
        out_shape=_f32(N_PAD, HP),
    )(x_pad, wgc_pad, degp)


def _tc_x1_body(nid_ref, deg_ref, acc_ref, y_ref, bgc_ref, w1c_ref, bd1_ref,
                x1_ref, cvec_ref):
    R = 1280
    i = pl.program_id(0)
    nid = nid_ref[0]
    deg = deg_ref[0, :, 0:1] + deg_ref[1, :, 0:1] + 1.0
    s = lax.rsqrt(deg)
    acc = acc_ref[0] + acc_ref[1]
    x1 = jnp.maximum(s * (acc + y_ref[...]) + bgc_ref[...], 0.0)
    x1_ref[...] = x1
    q = jnp.dot(x1, w1c_ref[...], preferred_element_type=jnp.float32)

    @pl.when(i == nid // R)
    def _():
        lid = nid - (nid // R) * R
        rowids = lax.broadcasted_iota(jnp.int32, (R, 1), 0)
        sel = jnp.where(rowids == lid, 1.0, 0.0)
        cvec_ref[...] = jnp.sum(q * sel, axis=0, keepdims=True) + bd1_ref[...]


def _tc_x1(nid_arr, degp, accp, y, bgc_pad, w1c_pad, bd1):
    R = 1280
    grid_spec = pltpu.PrefetchScalarGridSpec(
        num_scalar_prefetch=1,
        grid=(N_PAD // R,),
        in_specs=[
            pl.BlockSpec((NC, R, HP), lambda i, nid: (0, i, 0)),
            pl.BlockSpec((NC, R, HP), lambda i, nid: (0, i, 0)),
            pl.BlockSpec((R, HP), lambda i, nid: (i, 0)),
            pl.BlockSpec((1, HP), lambda i, nid: (0, 0)),
            pl.BlockSpec((HP, DEC_H), lambda i, nid: (0, 0)),
            pl.BlockSpec((1, DEC_H), lambda i, nid: (0, 0)),
        ],
        out_specs=[
            pl.BlockSpec((R, HP), lambda i, nid: (i, 0)),
            pl.BlockSpec((1, DEC_H), lambda i, nid: (0, 0)),
        ],
    )
    return pl.pallas_call(
        _tc_x1_body,
        grid_spec=grid_spec,
        out_shape=[_f32(N_PAD, HP), _f32(1, DEC_H)],
    )(nid_arr, degp, accp, y, bgc_pad, w1c_pad, bd1)


def _tc_dec_body(za_ref, zb_ref, cvec_ref, w1a_ref, w1b_ref, w2s_ref, b2_ref,
                 u_ref, out_ref):
    cv = cvec_ref[...]
    cv4 = jnp.concatenate([cv, cv, cv, cv], axis=1)
    h = (jnp.dot(za_ref[...], w1a_ref[...], preferred_element_type=jnp.float32)
         + jnp.dot(zb_ref[...], w1b_ref[...], preferred_element_type=jnp.float32)
         + cv4)
    h = jnp.maximum(h, 0.0)
    wt = lax.dot_general(w2s_ref[...], h, (((1,), (1,)), ((), ())),
                         preferred_element_type=jnp.float32)
    w = wt + b2_ref[0, 0]
    bias = 0.0001
    u = u_ref[...]
    eps = (bias - (1.0 - bias)) * u + (1.0 - bias)
    gate = jnp.log(eps) - jnp.log(1.0 - eps) + w
    out_ref[...] = jax.nn.sigmoid(gate)


def _tc_decoder(za, zb, cvec, w1a4, w1b4, w2s, b2r, u4):
    BR = 512
    return pl.pallas_call(
        _tc_dec_body,
        grid=(NE4 // BR,),
        in_specs=[
            pl.BlockSpec((BR, D_IN), lambda i: (i, 0)),
            pl.BlockSpec((BR, D_IN), lambda i: (i, 0)),
            pl.BlockSpec((1, DEC_H), lambda i: (0, 0)),
            pl.BlockSpec((D_IN, 4 * DEC_H), lambda i: (0, 0)),
            pl.BlockSpec((D_IN, 4 * DEC_H), lambda i: (0, 0)),
            pl.BlockSpec((4, 4 * DEC_H), lambda i: (0, 0)),
            pl.BlockSpec((1, 1), lambda i: (0, 0)),
            pl.BlockSpec((4, BR), lambda i: (0, i)),
        ],
        out_specs=pl.BlockSpec((4, BR), lambda i: (0, i)),
        out_shape=_f32(4, NE4),
    )(za, zb, cvec, w1a4, w1b4, w2s, b2r, u4)


def kernel(x, edge_index, node_id, W_gc, b_gc, W_d1, b_d1, W_d2, b_d2):
    f32 = jnp.float32
    x_pad = jnp.zeros((N_PAD, D_IN), f32).at[:N].set(x)
    wgc_pad = jnp.zeros((D_IN, HP), f32).at[:, :H].set(W_gc)
    bgc_pad = jnp.zeros((1, HP), f32).at[0, :H].set(b_gc)
    w1a4 = jnp.zeros((D_IN, 4 * DEC_H), f32)
    w1b4 = jnp.zeros((D_IN, 4 * DEC_H), f32)
    w2s = jnp.zeros((4, 4 * DEC_H), f32)
    for j in range(4):
        w1a4 = w1a4.at[j * HP:j * HP + H, j * DEC_H:(j + 1) * DEC_H].set(W_d1[:H])
        w1b4 = w1b4.at[j * HP:j * HP + H, j * DEC_H:(j + 1) * DEC_H].set(W_d1[H:2 * H])
        w2s = w2s.at[j, j * DEC_H:(j + 1) * DEC_H].set(W_d2[:, 0])
    w1c_pad = jnp.zeros((HP, DEC_H), f32).at[:H].set(W_d1[2 * H:])
    bd1 = b_d1.reshape(1, DEC_H).astype(f32)
    b2r = b_d2.reshape(1, 1).astype(f32)
    nrows = TOT_CHUNKS + SLACK
    row3 = jnp.full((nrows * CHUNK,), N, jnp.int32).at[:E].set(
        edge_index[0]).reshape(nrows, CHUNK)
    col3 = jnp.full((nrows * CHUNK,), N, jnp.int32).at[:E].set(
        edge_index[1]).reshape(nrows, CHUNK)
    ones_t = jnp.zeros((CHUNK, HP), f32).at[:, 0].set(1.0)
    zeros_t = jnp.zeros((N_PAD, HP), f32)
    u4 = jnp.asarray(_U4)
    nid_arr = jnp.asarray(node_id, jnp.int32).reshape(1)

    degp = _sc_degree(col3, ones_t, zeros_t)
    y = _tc_y(x_pad, wgc_pad, degp)
    accp = _sc_scatter(row3, col3, y, zeros_t)
    x1, cvec = _tc_x1(nid_arr, degp, accp, y, bgc_pad, w1c_pad, bd1)
    za, zb = _sc_gather(row3, col3, x1)
    za4 = za.reshape(NE4, D_IN)
    zb4 = zb.reshape(NE4, D_IN)
    out4 = _tc_decoder(za4, zb4, cvec, w1a4, w1b4, w2s, b2r, u4)
    return out4.T.reshape(E_PAD, 1)[:E]


# speedup vs baseline: 1.0985x; 1.0985x over previous
import functools

import jax
import jax.numpy as jnp
from jax import lax
from jax.experimental import pallas as pl
from jax.experimental.pallas import tpu as pltpu
from jax.experimental.pallas import tpu_sc as plsc

N = 10000
E = 320000
D_IN = 128
H = 20
HP = 32
DEC_H = 64
N_PAD = 10240
NC = 2
NS = 16
NW = NC * NS
CHUNK = 128
NCH = 80
SUP = 4
TOT_CHUNKS = NW * NCH
E_PAD = TOT_CHUNKS * CHUNK
NE4 = E_PAD // 4
ROWS_PER_TILE = N_PAD // NS

def _make_noise_u4():
    import numpy as np

    def rotl(x, r):
        return ((x << np.uint32(r)) | (x >> np.uint32(32 - r))).astype(np.uint32)

    rotations = ((13, 15, 26, 6), (17, 29, 16, 24))
    ks0, ks1 = np.uint32(0), np.uint32(42)
    ks2 = np.uint32(ks0 ^ ks1 ^ np.uint32(0x1BD11BDA))
    ks = (ks0, ks1, ks2)
    i = np.arange(E, dtype=np.uint64)
    x0 = (i >> np.uint64(32)).astype(np.uint32)
    x1 = (i & np.uint64(0xFFFFFFFF)).astype(np.uint32)
    x0 = (x0 + ks0).astype(np.uint32)
    x1 = (x1 + ks1).astype(np.uint32)
    for r in range(5):
        for rot in rotations[r % 2]:
            x0 = (x0 + x1).astype(np.uint32)
            x1 = rotl(x1, rot)
            x1 = (x1 ^ x0).astype(np.uint32)
        x0 = (x0 + ks[(r + 1) % 3]).astype(np.uint32)
        x1 = (x1 + ks[(r + 2) % 3] + np.uint32(r + 1)).astype(np.uint32)
    bits = (x0 ^ x1).astype(np.uint32)
    floats = (bits >> np.uint32(9)) | np.uint32(0x3F800000)
    u = floats.view(np.float32) - np.float32(1.0)
    u_pad = np.full((E_PAD,), 0.5, np.float32)
    u_pad[:E] = u
    return np.ascontiguousarray(u_pad.reshape(NE4, 4).T)


_U4 = _make_noise_u4()


def _f32(*shape):
    return jax.ShapeDtypeStruct(shape, jnp.float32)


@functools.lru_cache(maxsize=None)
def _mesh():
    return plsc.VectorSubcoreMesh(core_axis_name="c", subcore_axis_name="s",
                                  num_cores=NC, num_subcores=NS)


def _pipe2(nch, start_fn, finish_fn):
    start_fn(0, 0)

    def body(i, _):
        j0 = 2 * i
        j1 = j0 + 1
        start_fn(j1, 1)
        finish_fn(j0, 0)

        @pl.when(j1 + 1 < nch)
        def _():
            start_fn(j1 + 1, 0)

        finish_fn(j1, 1)
        return 0

    lax.fori_loop(0, nch // 2, body, 0)


def _sc_degree(col3, ones_t, zeros_t):
    return _sc_degree_kernel()(col3, ones_t, zeros_t)


@functools.lru_cache(maxsize=None)
def _sc_degree_kernel():
    return functools.partial(
        pl.kernel,
        out_type=_f32(NC, N_PAD, HP),
        mesh=_mesh(),
        compiler_params=pltpu.CompilerParams(use_tc_tiling_on_sc=False),
        scratch_types=[
            pltpu.VMEM((NCH, CHUNK), jnp.int32),
            pltpu.VMEM((CHUNK, HP), jnp.float32),
            pltpu.VMEM_SHARED((N_PAD, HP), jnp.float32),
        ],
    )(_sc_degree_body)


def _sc_degree_body(col_hbm, ones_hbm, zeros_hbm, out_hbm, cidx, ones_v, hist_sh):
    cid = lax.axis_index("c")
    sid = lax.axis_index("s")
    wid = cid * NS + sid
    pltpu.sync_copy(zeros_hbm.at[pl.ds(sid * ROWS_PER_TILE, ROWS_PER_TILE)],
                    hist_sh.at[pl.ds(sid * ROWS_PER_TILE, ROWS_PER_TILE)])
    pltpu.sync_copy(ones_hbm, ones_v)
    pltpu.sync_copy(col_hbm.at[wid], cidx)
    plsc.subcore_barrier()

    def body(j, _):
        pltpu.sync_copy(ones_v, hist_sh.at[cidx.at[j]], add=True)
        return 0

    lax.fori_loop(0, NCH, body, 0)
    plsc.subcore_barrier()
    pltpu.sync_copy(hist_sh.at[pl.ds(sid * ROWS_PER_TILE, ROWS_PER_TILE)],
                    out_hbm.at[cid, pl.ds(sid * ROWS_PER_TILE, ROWS_PER_TILE)])


def _sc_scatter(row3, col3, y, zeros_t):
    return _sc_scatter_kernel()(row3, col3, y, zeros_t)


@functools.lru_cache(maxsize=None)
def _sc_scatter_kernel():
    return functools.partial(
        pl.kernel,
        out_type=_f32(NC, N_PAD, HP),
        mesh=_mesh(),
        compiler_params=pltpu.CompilerParams(use_tc_tiling_on_sc=False),
        scratch_types=[
            pltpu.VMEM((NCH, CHUNK), jnp.int32),
            pltpu.VMEM((NCH, CHUNK), jnp.int32),
            pltpu.VMEM((SUP * CHUNK, HP), jnp.float32),
            pltpu.VMEM((SUP * CHUNK, HP), jnp.float32),
            pltpu.VMEM_SHARED((N_PAD, HP), jnp.float32),
            pltpu.SemaphoreType.DMA,
            pltpu.SemaphoreType.DMA,
        ],
    )(_sc_scatter_body)


def _sc_scatter_body(row_hbm, col_hbm, y_hbm, zeros_hbm, out_hbm,
                     ridx, cidx, m0, m1, acc_sh, sg0, sg1):
    cid = lax.axis_index("c")
    sid = lax.axis_index("s")
    wid = cid * NS + sid
    pltpu.sync_copy(zeros_hbm.at[pl.ds(sid * ROWS_PER_TILE, ROWS_PER_TILE)],
                    acc_sh.at[pl.ds(sid * ROWS_PER_TILE, ROWS_PER_TILE)])
    pltpu.sync_copy(row_hbm.at[wid], ridx)
    pltpu.sync_copy(col_hbm.at[wid], cidx)
    plsc.subcore_barrier()

    bufs = (m0, m1)
    sems = (sg0, sg1)

    def start(su, slot):
        for c in range(SUP):
            pltpu.async_copy(y_hbm.at[ridx.at[su * SUP + c]],
                             bufs[slot].at[pl.ds(c * CHUNK, CHUNK)],
                             sems[slot])

    def finish(su, slot):
        pltpu.make_async_copy(y_hbm, bufs[slot], sems[slot]).wait()
        for c in range(SUP):
            pltpu.sync_copy(bufs[slot].at[pl.ds(c * CHUNK, CHUNK)],
                            acc_sh.at[cidx.at[su * SUP + c]], add=True)

    _pipe2(NCH // SUP, start, finish)
    plsc.subcore_barrier()
    pltpu.sync_copy(acc_sh.at[pl.ds(sid * ROWS_PER_TILE, ROWS_PER_TILE)],
                    out_hbm.at[cid, pl.ds(sid * ROWS_PER_TILE, ROWS_PER_TILE)])


def _sc_gather(row3, col3, x1):
    return _sc_gather_kernel()(row3, col3, x1)


@functools.lru_cache(maxsize=None)
def _sc_gather_kernel():
    return functools.partial(
        pl.kernel,
        out_type=(_f32(E_PAD, HP), _f32(E_PAD, HP)),
        mesh=_mesh(),
        compiler_params=pltpu.CompilerParams(use_tc_tiling_on_sc=False),
        scratch_types=[
            pltpu.VMEM((NCH, CHUNK), jnp.int32),
            pltpu.VMEM((NCH, CHUNK), jnp.int32),
            pltpu.VMEM((SUP * CHUNK, HP), jnp.float32),
            pltpu.VMEM((SUP * CHUNK, HP), jnp.float32),
            pltpu.VMEM((SUP * CHUNK, HP), jnp.float32),
            pltpu.VMEM((SUP * CHUNK, HP), jnp.float32),
            pltpu.SemaphoreType.DMA,
            pltpu.SemaphoreType.DMA,
            pltpu.SemaphoreType.DMA,
            pltpu.SemaphoreType.DMA,
            pltpu.SemaphoreType.DMA,
            pltpu.SemaphoreType.DMA,
            pltpu.SemaphoreType.DMA,
            pltpu.SemaphoreType.DMA,
        ],
    )(_sc_gather_body)


def _sc_gather_body(row_hbm, col_hbm, x1_hbm, za_hbm, zb_hbm,
                    ridx, cidx, a0, a1, b0, b1,
                    sa0, sa1, sb0, sb1, swa0, swa1, swb0, swb1):
    cid = lax.axis_index("c")
    sid = lax.axis_index("s")
    wid = cid * NS + sid
    pltpu.sync_copy(row_hbm.at[wid], ridx)
    pltpu.sync_copy(col_hbm.at[wid], cidx)

    abufs = (a0, a1)
    bbufs = (b0, b1)
    asems = (sa0, sa1)
    bsems = (sb0, sb1)
    wasems = (swa0, swa1)
    wbsems = (swb0, swb1)
    sup_rows = SUP * CHUNK

    def start(su, slot):
        @pl.when(su >= 2)
        def _():
            pltpu.make_async_copy(abufs[slot], za_hbm.at[pl.ds(0, sup_rows)],
                                  wasems[slot]).wait()
            pltpu.make_async_copy(bbufs[slot], zb_hbm.at[pl.ds(0, sup_rows)],
                                  wbsems[slot]).wait()
        for c in range(SUP):
            pltpu.async_copy(x1_hbm.at[ridx.at[su * SUP + c]],
                             abufs[slot].at[pl.ds(c * CHUNK, CHUNK)],
                             asems[slot])
            pltpu.async_copy(x1_hbm.at[cidx.at[su * SUP + c]],
                             bbufs[slot].at[pl.ds(c * CHUNK, CHUNK)],
                             bsems[slot])

    def finish(su, slot):
        pltpu.make_async_copy(x1_hbm, abufs[slot], asems[slot]).wait()
        pltpu.make_async_copy(x1_hbm, bbufs[slot], bsems[slot]).wait()
        base = wid * NCH * CHUNK + su * sup_rows
        pltpu.async_copy(abufs[slot], za_hbm.at[pl.ds(base, sup_rows)],
                         wasems[slot])
        pltpu.async_copy(bbufs[slot], zb_hbm.at[pl.ds(base, sup_rows)],
                         wbsems[slot])

    _pipe2(NCH // SUP, start, finish)
    for slot in (0, 1):
        pltpu.make_async_copy(abufs[slot], za_hbm.at[pl.ds(0, sup_rows)],
                              wasems[slot]).wait()
        pltpu.make_async_copy(bbufs[slot], zb_hbm.at[pl.ds(0, sup_rows)],
                              wbsems[slot]).wait()


def _tc_y_body(x_ref, w_ref, deg_ref, y_ref):
    deg = deg_ref[0, :, 0:1] + deg_ref[1, :, 0:1] + 1.0
    s = lax.rsqrt(deg)
    xw = jnp.dot(x_ref[...], w_ref[...], preferred_element_type=jnp.float32)
    y_ref[...] = xw * s


def _tc_y(x_pad, wgc_pad, degp):
    R = 1280
    return pl.pallas_call(
        _tc_y_body,
        grid=(N_PAD // R,),
        in_specs=[
            pl.BlockSpec((R, D_IN), lambda i: (i, 0)),
            pl.BlockSpec((D_IN, HP), lambda i: (0, 0)),
            pl.BlockSpec((NC, R, HP), lambda i: (0, i, 0)),
        ],
        out_specs=pl.BlockSpec((R, HP), lambda i: (i, 0)),
        out_shape=_f32(N_PAD, HP),
    )(x_pad, wgc_pad, degp)


def _tc_x1_body(nid_ref, deg_ref, acc_ref, y_ref, bgc_ref, w1c_ref, bd1_ref,
                x1_ref, cvec_ref):
    R = 1280
    i = pl.program_id(0)
    nid = nid_ref[0]
    deg = deg_ref[0, :, 0:1] + deg_ref[1, :, 0:1] + 1.0
    s = lax.rsqrt(deg)
    acc = acc_ref[0] + acc_ref[1]
    x1 = jnp.maximum(s * (acc + y_ref[...]) + bgc_ref[...], 0.0)
    x1_ref[...] = x1
    q = jnp.dot(x1, w1c_ref[...], preferred_element_type=jnp.float32)

    @pl.when(i == nid // R)
    def _():
        lid = nid - (nid // R) * R
        rowids = lax.broadcasted_iota(jnp.int32, (R, 1), 0)
        sel = jnp.where(rowids == lid, 1.0, 0.0)
        cvec_ref[...] = jnp.sum(q * sel, axis=0, keepdims=True) + bd1_ref[...]


def _tc_x1(nid_arr, degp, accp, y, bgc_pad, w1c_pad, bd1):
    R = 1280
    grid_spec = pltpu.PrefetchScalarGridSpec(
        num_scalar_prefetch=1,
        grid=(N_PAD // R,),
        in_specs=[
            pl.BlockSpec((NC, R, HP), lambda i, nid: (0, i, 0)),
            pl.BlockSpec((NC, R, HP), lambda i, nid: (0, i, 0)),
            pl.BlockSpec((R, HP), lambda i, nid: (i, 0)),
            pl.BlockSpec((1, HP), lambda i, nid: (0, 0)),
            pl.BlockSpec((HP, DEC_H), lambda i, nid: (0, 0)),
            pl.BlockSpec((1, DEC_H), lambda i, nid: (0, 0)),
        ],
        out_specs=[
            pl.BlockSpec((R, HP), lambda i, nid: (i, 0)),
            pl.BlockSpec((1, DEC_H), lambda i, nid: (0, 0)),
        ],
    )
    return pl.pallas_call(
        _tc_x1_body,
        grid_spec=grid_spec,
        out_shape=[_f32(N_PAD, HP), _f32(1, DEC_H)],
    )(nid_arr, degp, accp, y, bgc_pad, w1c_pad, bd1)


def _tc_dec_body(za_ref, zb_ref, cvec_ref, w1a_ref, w1b_ref, w2s_ref, b2_ref,
                 u_ref, out_ref):
    cv = cvec_ref[...]
    cv4 = jnp.concatenate([cv, cv, cv, cv], axis=1)
    h = (jnp.dot(za_ref[...], w1a_ref[...], preferred_element_type=jnp.float32)
         + jnp.dot(zb_ref[...], w1b_ref[...], preferred_element_type=jnp.float32)
         + cv4)
    h = jnp.maximum(h, 0.0)
    wt = lax.dot_general(w2s_ref[...], h, (((1,), (1,)), ((), ())),
                         preferred_element_type=jnp.float32)
    w = wt + b2_ref[0, 0]
    bias = 0.0001
    u = u_ref[...]
    eps = (bias - (1.0 - bias)) * u + (1.0 - bias)
    gate = jnp.log(eps) - jnp.log(1.0 - eps) + w
    out_ref[...] = jax.nn.sigmoid(gate)


def _tc_decoder(za, zb, cvec, w1a4, w1b4, w2s, b2r, u4):
    BR = 512
    return pl.pallas_call(
        _tc_dec_body,
        grid=(NE4 // BR,),
        in_specs=[
            pl.BlockSpec((BR, D_IN), lambda i: (i, 0)),
            pl.BlockSpec((BR, D_IN), lambda i: (i, 0)),
            pl.BlockSpec((1, DEC_H), lambda i: (0, 0)),
            pl.BlockSpec((D_IN, 4 * DEC_H), lambda i: (0, 0)),
            pl.BlockSpec((D_IN, 4 * DEC_H), lambda i: (0, 0)),
            pl.BlockSpec((4, 4 * DEC_H), lambda i: (0, 0)),
            pl.BlockSpec((1, 1), lambda i: (0, 0)),
            pl.BlockSpec((4, BR), lambda i: (0, i)),
        ],
        out_specs=pl.BlockSpec((4, BR), lambda i: (0, i)),
        out_shape=_f32(4, NE4),
    )(za, zb, cvec, w1a4, w1b4, w2s, b2r, u4)


def kernel(x, edge_index, node_id, W_gc, b_gc, W_d1, b_d1, W_d2, b_d2):
    f32 = jnp.float32
    x_pad = jnp.zeros((N_PAD, D_IN), f32).at[:N].set(x)
    wgc_pad = jnp.zeros((D_IN, HP), f32).at[:, :H].set(W_gc)
    bgc_pad = jnp.zeros((1, HP), f32).at[0, :H].set(b_gc)
    w1a4 = jnp.zeros((D_IN, 4 * DEC_H), f32)
    w1b4 = jnp.zeros((D_IN, 4 * DEC_H), f32)
    w2s = jnp.zeros((4, 4 * DEC_H), f32)
    for j in range(4):
        w1a4 = w1a4.at[j * HP:j * HP + H, j * DEC_H:(j + 1) * DEC_H].set(W_d1[:H])
        w1b4 = w1b4.at[j * HP:j * HP + H, j * DEC_H:(j + 1) * DEC_H].set(W_d1[H:2 * H])
        w2s = w2s.at[j, j * DEC_H:(j + 1) * DEC_H].set(W_d2[:, 0])
    w1c_pad = jnp.zeros((HP, DEC_H), f32).at[:H].set(W_d1[2 * H:])
    bd1 = b_d1.reshape(1, DEC_H).astype(f32)
    b2r = b_d2.reshape(1, 1).astype(f32)
    row3 = jnp.full((E_PAD,), N, jnp.int32).at[:E].set(
        edge_index[0]).reshape(NW, NCH, CHUNK)
    col3 = jnp.full((E_PAD,), N, jnp.int32).at[:E].set(
        edge_index[1]).reshape(NW, NCH, CHUNK)
    ones_t = jnp.zeros((CHUNK, HP), f32).at[:, 0].set(1.0)
    zeros_t = jnp.zeros((N_PAD, HP), f32)
    u4 = jnp.asarray(_U4)
    nid_arr = jnp.asarray(node_id, jnp.int32).reshape(1)

    degp = _sc_degree(col3, ones_t, zeros_t)
    y = _tc_y(x_pad, wgc_pad, degp)
    accp = _sc_scatter(row3, col3, y, zeros_t)
    x1, cvec = _tc_x1(nid_arr, degp, accp, y, bgc_pad, w1c_pad, bd1)
    za, zb = _sc_gather(row3, col3, x1)
    za4 = za.reshape(NE4, D_IN)
    zb4 = zb.reshape(NE4, D_IN)
    out4 = _tc_decoder(za4, zb4, cvec, w1a4, w1b4, w2s, b2r, u4)
    return out4.T.reshape(E_PAD, 1)[:E]
